# baseline (speedup 1.0000x reference)
import functools

import jax
import jax.numpy as jnp
from jax import lax
from jax.experimental import pallas as pl
from jax.experimental.pallas import tpu as pltpu
from jax.experimental.pallas import tpu_sc as plsc

_PS_DIM = 64
_NUM_WORDS = 16
_IDXW = 128
_LANES = 16


@functools.lru_cache(maxsize=None)
def _gather_call(total_rows, chunk):
    info = plsc.get_sparse_core_info()
    nw = info.num_cores * info.num_subcores
    per_w = total_rows // nw
    n_chunks = per_w // chunk
    npair = chunk // 2
    n_t = npair // _IDXW
    assert per_w * nw == total_rows and n_chunks * chunk == per_w
    assert n_chunks % 2 == 0 and n_t * _IDXW == npair

    mesh = plsc.VectorSubcoreMesh(core_axis_name="c", subcore_axis_name="s")

    @functools.partial(
        pl.kernel,
        mesh=mesh,
        compiler_params=pltpu.CompilerParams(needs_layout_passes=False),
        out_type=jax.ShapeDtypeStruct((total_rows // 2, 2 * _PS_DIM),
                                      jnp.float32),
        scratch_types=[
            pltpu.VMEM((chunk,), jnp.int32),
            pltpu.VMEM((chunk,), jnp.int32),
            pltpu.VMEM((npair,), jnp.int32),
            pltpu.VMEM((npair,), jnp.int32),
            pltpu.VMEM((npair, 2 * _PS_DIM), jnp.float32),
            pltpu.VMEM((npair, 2 * _PS_DIM), jnp.float32),
            pltpu.SemaphoreType.DMA,
            pltpu.SemaphoreType.DMA,
            pltpu.SemaphoreType.DMA,
            pltpu.SemaphoreType.DMA,
            pltpu.SemaphoreType.DMA,
            pltpu.SemaphoreType.DMA,
        ],
    )
    def k(table_hbm, idx_hbm, out_hbm, idx_v0, idx_v1, pidx_v0, pidx_v1,
          rows_v0, rows_v1, isem0, isem1, gsem0, gsem1, osem0, osem1):
        wid = lax.axis_index("s") * info.num_cores + lax.axis_index("c")
        base = wid * per_w
        pbase = base // 2
        idx_bufs = (idx_v0, idx_v1)
        pidx_bufs = (pidx_v0, pidx_v1)
        rows_bufs = (rows_v0, rows_v1)
        isems = (isem0, isem1)
        gsems = (gsem0, gsem1)
        osems = (osem0, osem1)
        lane2 = lax.iota(jnp.int32, _LANES) * 2

        def idx_start(g, par):
            return pltpu.make_async_copy(
                idx_hbm.at[pl.ds(base + g * chunk, chunk)],
                idx_bufs[par],
                isems[par],
            )

        def fuse_pairs(par):
            idx_v = idx_bufs[par]
            pidx_v = pidx_bufs[par]

            def pb(i, c):
                b0 = i * (2 * _LANES)
                ev = plsc.load_gather(idx_v, [lane2 + b0])
                od = plsc.load_gather(idx_v, [lane2 + (b0 + 1)])
                pidx_v[pl.ds(i * _LANES, _LANES)] = ev * _NUM_WORDS + od
                return c

            lax.fori_loop(0, npair // _LANES, pb, 0, unroll=False)

        def gathers(par):
            for t in range(n_t):
                pltpu.make_async_copy(
                    table_hbm.at[pidx_bufs[par].at[pl.ds(t * _IDXW, _IDXW)]],
                    rows_bufs[par].at[pl.ds(t * _IDXW, _IDXW)],
                    gsems[par],
                ).start()

        def drain(par):
            for t in range(n_t):
                pltpu.make_async_copy(
                    table_hbm.at[pidx_bufs[par].at[pl.ds(t * _IDXW, _IDXW)]],
                    rows_bufs[par].at[pl.ds(t * _IDXW, _IDXW)],
                    gsems[par],
                ).wait()

        def out_copy(g, par):
            return pltpu.make_async_copy(
                rows_bufs[par],
                out_hbm.at[pl.ds(pl.multiple_of(pbase + g * npair, npair),
                                 npair)],
                osems[par],
            )

        idx_start(0, 0).start()
        idx_start(1, 1).start()
        idx_start(0, 0).wait()
        fuse_pairs(0)
        gathers(0)
        drain(0)
        out_copy(0, 0).start()

        def pair_body(g2, c):
            g = g2 * 2 + 1
            idx_start(g, 1).wait()
            idx_start(g + 1, 0).start()
            fuse_pairs(1)

            @pl.when(g2 > 0)
            def _():
                out_copy(g - 2, 1).wait()

            gathers(1)
            drain(1)
            out_copy(g, 1).start()
            idx_start(g + 1, 0).wait()

            @pl.when(g + 2 < n_chunks)
            def _():
                idx_start(g + 2, 1).start()

            fuse_pairs(0)
            out_copy(g - 1, 0).wait()
            gathers(0)
            drain(0)
            out_copy(g + 1, 0).start()
            return c

        lax.fori_loop(0, (n_chunks - 2) // 2, pair_body, 0, unroll=False)

        g_last = n_chunks - 1
        idx_start(g_last, 1).wait()
        fuse_pairs(1)
        out_copy(g_last - 2, 1).wait()
        gathers(1)
        drain(1)
        out_copy(g_last, 1).start()
        out_copy(g_last - 1, 0).wait()
        out_copy(g_last, 1).wait()

    return k


def kernel(batch_rgn_sqn, encoding):
    b, l = batch_rgn_sqn.shape
    table16 = encoding[:_NUM_WORDS, :_PS_DIM]
    pair_table = jnp.concatenate(
        [jnp.repeat(table16, _NUM_WORDS, axis=0),
         jnp.tile(table16, (_NUM_WORDS, 1))], axis=1)
    idx = batch_rgn_sqn.reshape(-1).astype(jnp.int32)
    out = _gather_call(b * l, 512)(pair_table, idx)
    return out.reshape(b, l, _PS_DIM)


# speedup vs baseline: 2.2019x; 2.2019x over previous
import functools

import jax
import jax.numpy as jnp
from jax import lax
from jax.experimental import pallas as pl
from jax.experimental.pallas import tpu as pltpu
from jax.experimental.pallas import tpu_sc as plsc

_PS_DIM = 64
_NUM_WORDS = 16
_LANES = 16
_SKEW = 1041
_BB = 128
_RO = 8


@functools.lru_cache(maxsize=None)
def _gather_call(n_batch, seq_len):
    info = plsc.get_sparse_core_info()
    nw = info.num_cores * info.num_subcores
    n_oct = seq_len // _RO
    half = _RO // 2
    assert n_batch == _BB * nw and n_oct * _RO == seq_len

    mesh = plsc.VectorSubcoreMesh(core_axis_name="c", subcore_axis_name="s")

    @functools.partial(
        pl.kernel,
        mesh=mesh,
        compiler_params=pltpu.CompilerParams(
            needs_layout_passes=False, use_tc_tiling_on_sc=True),
        out_type=jax.ShapeDtypeStruct((seq_len, _PS_DIM, n_batch),
                                      jnp.float32),
        scratch_types=[
            pltpu.VMEM((_NUM_WORDS * _SKEW,), jnp.float32),
            pltpu.VMEM((_RO, _BB), jnp.int32),
            pltpu.VMEM((half, _PS_DIM, _BB), jnp.float32),
            pltpu.VMEM((half, _PS_DIM, _BB), jnp.float32),
            pltpu.SemaphoreType.DMA,
            pltpu.SemaphoreType.DMA,
        ],
    )
    def k(skew_hbm, idxt_hbm, out_hbm, skew_v, idxs_v, buf0, buf1,
          osem0, osem1):
        wid = lax.axis_index("s") * info.num_cores + lax.axis_index("c")
        b0 = pl.multiple_of(wid * _BB, _BB)
        bufs = (buf0, buf1)
        osems = (osem0, osem1)
        lane = lax.iota(jnp.int32, _LANES)
        skew_base = lane * _SKEW

        pltpu.sync_copy(skew_hbm, skew_v)

        def out_dma(o, h):
            r0 = o * _RO + h * half
            return pltpu.make_async_copy(
                bufs[h],
                out_hbm.at[pl.ds(r0, half), :, pl.ds(b0, _BB)],
                osems[h],
            )

        def compose(h):
            def grp(v, c):
                l0 = v * _LANES
                for i in range(half):
                    idxv = idxs_v[h * half + i, pl.ds(l0, _LANES)]
                    fb = skew_base + idxv * _PS_DIM
                    for c0 in range(_PS_DIM):
                        val = plsc.load_gather(skew_v, [fb])
                        bufs[h][i, c0, pl.ds(l0, _LANES)] = val
                        if c0 != _PS_DIM - 1:
                            fb = fb + 1
                return c

            lax.fori_loop(0, _BB // _LANES, grp, 0, unroll=False)

        def o_body(o, c):
            r0 = pl.multiple_of(o * _RO, _RO)
            pltpu.sync_copy(idxt_hbm.at[pl.ds(r0, _RO), pl.ds(b0, _BB)],
                            idxs_v)
            for h in range(2):
                @pl.when(o > 0)
                def _():
                    out_dma(0, h).wait()

                compose(h)
                out_dma(o, h).start()
            return c

        lax.fori_loop(0, n_oct, o_body, 0, unroll=False)
        out_dma(0, 0).wait()
        out_dma(0, 1).wait()

    return k


def kernel(batch_rgn_sqn, encoding):
    b, l = batch_rgn_sqn.shape
    table_flat = encoding[:_NUM_WORDS, :_PS_DIM].reshape(1, -1)
    skew = jnp.pad(jnp.tile(table_flat, (_NUM_WORDS, 1)),
                   ((0, 0), (0, _SKEW - _NUM_WORDS * _PS_DIM))).reshape(-1)
    idxt = batch_rgn_sqn.T.astype(jnp.int32)
    out_t = _gather_call(b, l)(skew, idxt)
    return jnp.transpose(out_t, (2, 0, 1))


# speedup vs baseline: 2.2076x; 1.0026x over previous
import functools

import jax
import jax.numpy as jnp
from jax import lax
from jax.experimental import pallas as pl
from jax.experimental.pallas import tpu as pltpu
from jax.experimental.pallas import tpu_sc as plsc

_PS_DIM = 64
_NUM_WORDS = 16
_LANES = 16
_SKEW = 1041
_BB = 128
_RO = 8


@functools.lru_cache(maxsize=None)
def _gather_call(n_batch, seq_len):
    info = plsc.get_sparse_core_info()
    nw = info.num_cores * info.num_subcores
    n_oct = seq_len // _RO
    half = _RO // 2
    assert n_batch == _BB * nw and n_oct * _RO == seq_len

    mesh = plsc.VectorSubcoreMesh(core_axis_name="c", subcore_axis_name="s")

    @functools.partial(
        pl.kernel,
        mesh=mesh,
        compiler_params=pltpu.CompilerParams(
            needs_layout_passes=False, use_tc_tiling_on_sc=True),
        out_type=jax.ShapeDtypeStruct((seq_len, _PS_DIM, n_batch),
                                      jnp.float32),
        scratch_types=[
            pltpu.VMEM((_NUM_WORDS * _SKEW,), jnp.float32),
            pltpu.VMEM((_RO, _BB), jnp.int32),
            pltpu.VMEM((half, _PS_DIM, _BB), jnp.float32),
            pltpu.VMEM((half, _PS_DIM, _BB), jnp.float32),
            pltpu.SemaphoreType.DMA,
            pltpu.SemaphoreType.DMA,
        ],
    )
    def k(skew_hbm, idxt_hbm, out_hbm, skew_v, idxs_v, buf0, buf1,
          osem0, osem1):
        wid = lax.axis_index("s") * info.num_cores + lax.axis_index("c")
        b0 = pl.multiple_of(wid * _BB, _BB)
        bufs = (buf0, buf1)
        osems = (osem0, osem1)
        lane = lax.iota(jnp.int32, _LANES)
        skew_base = lane * _SKEW

        pltpu.sync_copy(skew_hbm, skew_v)

        def out_dma(o, h):
            r0 = o * _RO + h * half
            return pltpu.make_async_copy(
                bufs[h],
                out_hbm.at[pl.ds(r0, half), :, pl.ds(b0, _BB)],
                osems[h],
            )

        def compose(h):
            def grp(v, c):
                l0 = v * _LANES
                for i in range(half):
                    idxv = idxs_v[h * half + i, pl.ds(l0, _LANES)]
                    fb0 = skew_base + idxv * _PS_DIM
                    for c0 in range(_PS_DIM):
                        val = plsc.load_gather(skew_v, [fb0 + c0])
                        bufs[h][i, c0, pl.ds(l0, _LANES)] = val
                return c

            lax.fori_loop(0, _BB // _LANES, grp, 0, unroll=False)

        def o_body(o, c):
            r0 = pl.multiple_of(o * _RO, _RO)
            pltpu.sync_copy(idxt_hbm.at[pl.ds(r0, _RO), pl.ds(b0, _BB)],
                            idxs_v)
            for h in range(2):
                @pl.when(o > 0)
                def _():
                    out_dma(0, h).wait()

                compose(h)
                out_dma(o, h).start()
            return c

        lax.fori_loop(0, n_oct, o_body, 0, unroll=False)
        out_dma(0, 0).wait()
        out_dma(0, 1).wait()

    return k


def kernel(batch_rgn_sqn, encoding):
    b, l = batch_rgn_sqn.shape
    table_flat = encoding[:_NUM_WORDS, :_PS_DIM].reshape(1, -1)
    skew = jnp.pad(jnp.tile(table_flat, (_NUM_WORDS, 1)),
                   ((0, 0), (0, _SKEW - _NUM_WORDS * _PS_DIM))).reshape(-1)
    idxt = batch_rgn_sqn.T.astype(jnp.int32)
    out_t = _gather_call(b, l)(skew, idxt)
    return jnp.transpose(out_t, (2, 0, 1))


# speedup vs baseline: 2.2580x; 1.0228x over previous
import functools

import jax
import jax.numpy as jnp
from jax import lax
from jax.experimental import pallas as pl
from jax.experimental.pallas import tpu as pltpu
from jax.experimental.pallas import tpu_sc as plsc

_PS_DIM = 64
_NUM_WORDS = 16
_LANES = 16
_SKEW = 1041
_BB = 128
_RO = 8


@functools.lru_cache(maxsize=None)
def _gather_call(n_batch, seq_len):
    info = plsc.get_sparse_core_info()
    nw = info.num_cores * info.num_subcores
    n_oct = seq_len // _RO
    half = _RO // 2
    assert n_batch == _BB * nw and n_oct * _RO == seq_len

    mesh = plsc.VectorSubcoreMesh(core_axis_name="c", subcore_axis_name="s")

    @functools.partial(
        pl.kernel,
        mesh=mesh,
        compiler_params=pltpu.CompilerParams(
            needs_layout_passes=False, use_tc_tiling_on_sc=True),
        out_type=jax.ShapeDtypeStruct((seq_len, _PS_DIM, n_batch),
                                      jnp.float32),
        scratch_types=[
            pltpu.VMEM((_NUM_WORDS * _SKEW,), jnp.float32),
            pltpu.VMEM((_RO, _BB), jnp.int32),
            pltpu.VMEM((half, _PS_DIM, _BB), jnp.float32),
            pltpu.VMEM((half, _PS_DIM, _BB), jnp.float32),
            pltpu.SemaphoreType.DMA,
            pltpu.SemaphoreType.DMA,
        ],
    )
    def k(skew_hbm, idxt_hbm, out_hbm, skew_v, idxs_v, buf0, buf1,
          osem0, osem1):
        wid = lax.axis_index("s") * info.num_cores + lax.axis_index("c")
        b0 = pl.multiple_of(wid * _BB, _BB)
        bufs = (buf0, buf1)
        osems = (osem0, osem1)
        lane = lax.iota(jnp.int32, _LANES)
        skew_base = lane * _SKEW

        pltpu.sync_copy(skew_hbm, skew_v)

        def out_dma(o, h):
            r0 = o * _RO + h * half
            return pltpu.make_async_copy(
                bufs[h],
                out_hbm.at[pl.ds(r0, half), :, pl.ds(b0, _BB)],
                osems[h],
            )

        def compose(h):
            def grp(v, c):
                l0 = v * _LANES
                fbs = []
                for i in range(half):
                    idxv = idxs_v[h * half + i, pl.ds(l0, _LANES)]
                    fbs.append(skew_base + idxv * _PS_DIM)
                for c0 in range(_PS_DIM):
                    for i in range(half):
                        val = plsc.load_gather(skew_v, [fbs[i] + c0])
                        bufs[h][i, c0, pl.ds(l0, _LANES)] = val
                return c

            lax.fori_loop(0, _BB // _LANES, grp, 0, unroll=False)

        def o_body(o, c):
            r0 = pl.multiple_of(o * _RO, _RO)
            pltpu.sync_copy(idxt_hbm.at[pl.ds(r0, _RO), pl.ds(b0, _BB)],
                            idxs_v)
            for h in range(2):
                @pl.when(o > 0)
                def _():
                    out_dma(0, h).wait()

                compose(h)
                out_dma(o, h).start()
            return c

        lax.fori_loop(0, n_oct, o_body, 0, unroll=False)
        out_dma(0, 0).wait()
        out_dma(0, 1).wait()

    return k


def kernel(batch_rgn_sqn, encoding):
    b, l = batch_rgn_sqn.shape
    table_flat = encoding[:_NUM_WORDS, :_PS_DIM].reshape(1, -1)
    skew = jnp.pad(jnp.tile(table_flat, (_NUM_WORDS, 1)),
                   ((0, 0), (0, _SKEW - _NUM_WORDS * _PS_DIM))).reshape(-1)
    idxt = batch_rgn_sqn.T.astype(jnp.int32)
    out_t = _gather_call(b, l)(skew, idxt)
    return jnp.transpose(out_t, (2, 0, 1))


# speedup vs baseline: 2.3292x; 1.0315x over previous
import functools

import jax
import jax.numpy as jnp
from jax import lax
from jax.experimental import pallas as pl
from jax.experimental.pallas import tpu as pltpu
from jax.experimental.pallas import tpu_sc as plsc

_PS_DIM = 64
_NUM_WORDS = 16
_LANES = 16
_SKEW = 1041
_BB = 128
_RO = 8


@functools.lru_cache(maxsize=None)
def _gather_call(n_batch, seq_len):
    info = plsc.get_sparse_core_info()
    nw = info.num_cores * info.num_subcores
    n_oct = seq_len // _RO
    half = _RO // 2
    assert n_batch == _BB * nw and n_oct * _RO == seq_len

    mesh = plsc.VectorSubcoreMesh(core_axis_name="c", subcore_axis_name="s")

    @functools.partial(
        pl.kernel,
        mesh=mesh,
        compiler_params=pltpu.CompilerParams(
            needs_layout_passes=False, use_tc_tiling_on_sc=True),
        out_type=jax.ShapeDtypeStruct((seq_len, _PS_DIM, n_batch),
                                      jnp.float32),
        scratch_types=[
            pltpu.VMEM((_NUM_WORDS * _SKEW,), jnp.float32),
            pltpu.VMEM((_RO, _BB), jnp.int32),
            pltpu.VMEM((_RO, _BB), jnp.int32),
            pltpu.VMEM((half, _PS_DIM, _BB), jnp.float32),
            pltpu.VMEM((half, _PS_DIM, _BB), jnp.float32),
            pltpu.SemaphoreType.DMA,
            pltpu.SemaphoreType.DMA,
            pltpu.SemaphoreType.DMA,
            pltpu.SemaphoreType.DMA,
        ],
    )
    def k(skew_hbm, idxt_hbm, out_hbm, skew_v, idxs_v0, idxs_v1, buf0, buf1,
          isem0, isem1, osem0, osem1):
        wid = lax.axis_index("s") * info.num_cores + lax.axis_index("c")
        b0 = pl.multiple_of(wid * _BB, _BB)
        bufs = (buf0, buf1)
        idx_bufs = (idxs_v0, idxs_v1)
        isems = (isem0, isem1)
        osems = (osem0, osem1)
        lane = lax.iota(jnp.int32, _LANES)
        skew_base = lane * _SKEW

        pltpu.sync_copy(skew_hbm, skew_v)

        def idx_dma(o, par):
            r0 = pl.multiple_of(o * _RO, _RO)
            return pltpu.make_async_copy(
                idxt_hbm.at[pl.ds(r0, _RO), pl.ds(b0, _BB)],
                idx_bufs[par],
                isems[par],
            )

        def out_dma(o, h):
            r0 = o * _RO + h * half
            return pltpu.make_async_copy(
                bufs[h],
                out_hbm.at[pl.ds(r0, half), :, pl.ds(b0, _BB)],
                osems[h],
            )

        def compose(par, h):
            idxs_v = idx_bufs[par]

            def grp(v, c):
                l0 = v * _LANES
                fbs = []
                for i in range(half):
                    idxv = idxs_v[h * half + i, pl.ds(l0, _LANES)]
                    fbs.append(skew_base + idxv * _PS_DIM)
                for c0 in range(_PS_DIM):
                    for i in range(half):
                        val = plsc.load_gather(skew_v, [fbs[i] + c0])
                        bufs[h][i, c0, pl.ds(l0, _LANES)] = val
                return c

            lax.fori_loop(0, _BB // _LANES, grp, 0, unroll=False)

        def process(o, par):
            idx_dma(o, par).wait()
            for h in range(2):
                @pl.when(o > 0)
                def _():
                    out_dma(0, h).wait()

                compose(par, h)
                out_dma(o, h).start()

        idx_dma(0, 0).start()

        def pair_body(g, c):
            o0 = g * 2
            idx_dma(o0 + 1, 1).start()
            process(o0, 0)
            idx_dma(o0 + 2, 0).start()
            process(o0 + 1, 1)
            return c

        lax.fori_loop(0, (n_oct - 1) // 2, pair_body, 0, unroll=False)
        process(n_oct - 1, 0)
        out_dma(0, 0).wait()
        out_dma(0, 1).wait()

    return k


def kernel(batch_rgn_sqn, encoding):
    b, l = batch_rgn_sqn.shape
    table_flat = encoding[:_NUM_WORDS, :_PS_DIM].reshape(1, -1)
    skew = jnp.pad(jnp.tile(table_flat, (_NUM_WORDS, 1)),
                   ((0, 0), (0, _SKEW - _NUM_WORDS * _PS_DIM))).reshape(-1)
    idxt = batch_rgn_sqn.T.astype(jnp.int32)
    out_t = _gather_call(b, l)(skew, idxt)
    return jnp.transpose(out_t, (2, 0, 1))
